# baseline (speedup 1.0000x reference)
import functools

import jax
import jax.numpy as jnp
from jax import lax
from jax.experimental import pallas as pl
from jax.experimental.pallas import tpu as pltpu
from jax.experimental.pallas import tpu_sc as plsc


@functools.lru_cache(maxsize=None)
def _make_gather(n_batch: int, n_seq: int, vocab: int, dmodel: int):
  n_total = n_batch * n_seq
  info = plsc.get_sparse_core_info()
  nw = info.num_cores * info.num_subcores
  assert n_total % nw == 0
  per_w = n_total // nw
  chunk = 800
  assert per_w % chunk == 0 and chunk % 8 == 0
  n_chunks = per_w // chunk

  assert per_w % n_seq == 0 and chunk % n_seq == 0
  rows_w = per_w // n_seq
  rows_c = chunk // n_seq

  mesh = plsc.VectorSubcoreMesh(core_axis_name="c", subcore_axis_name="s")

  @functools.partial(
      pl.kernel,
      mesh=mesh,
      compiler_params=pltpu.CompilerParams(use_tc_tiling_on_sc=False),
      out_type=jax.ShapeDtypeStruct((n_batch, n_seq, dmodel), jnp.float32),
      scratch_types=[
          pltpu.VMEM((per_w,), jnp.int32),
          pltpu.VMEM((2, rows_c, n_seq, dmodel), jnp.float32),
          pltpu.SemaphoreType.DMA,
          pltpu.SemaphoreType.DMA,
          pltpu.SemaphoreType.DMA,
          pltpu.SemaphoreType.DMA,
      ],
  )
  def k(idx_hbm, table_hbm, out3_hbm, idx_v, rows_v, sg0, sg1, sw0, sw1):
    wid = lax.axis_index("s") * info.num_cores + lax.axis_index("c")
    base = wid * per_w
    pltpu.sync_copy(idx_hbm.at[pl.ds(base, per_w)], idx_v)

    sg = (sg0, sg1)
    sw = (sw0, sw1)

    def start_gather(g):
      return [
          pltpu.async_copy(
              table_hbm.at[idx_v.at[pl.ds(g * chunk + r * n_seq, n_seq)]],
              rows_v.at[g % 2, r],
              sg[g % 2],
          )
          for r in range(rows_c)
      ]

    def start_wb(g):
      return pltpu.async_copy(
          rows_v.at[g % 2],
          out3_hbm.at[pl.ds(wid * rows_w + g * rows_c, rows_c)],
          sw[g % 2],
      )

    def wait_gather(hs):
      for h in hs:
        h.wait()

    gh = [None] * n_chunks
    wh = [None] * n_chunks
    gh[0] = start_gather(0)
    for g in range(n_chunks):
      if g >= 2:
        wh[g - 2].wait()
      if g >= 1:
        gh[g] = start_gather(g)
        wait_gather(gh[g - 1])
        wh[g - 1] = start_wb(g - 1)
    wait_gather(gh[n_chunks - 1])
    wh[n_chunks - 1] = start_wb(n_chunks - 1)
    wh[n_chunks - 2].wait()
    wh[n_chunks - 1].wait()

  return k


def kernel(x, weight):
  b, s = x.shape
  vocab, dmodel = weight.shape
  flat_idx = x.reshape(-1).astype(jnp.int32)
  return _make_gather(b, s, vocab, dmodel)(flat_idx, weight)


# speedup vs baseline: 1.1041x; 1.1041x over previous
import functools

import jax
import jax.numpy as jnp
from jax import lax
from jax.experimental import pallas as pl
from jax.experimental.pallas import tpu as pltpu
from jax.experimental.pallas import tpu_sc as plsc

_PAD = 128


@functools.lru_cache(maxsize=None)
def _make_gather(n_batch: int, n_seq: int, vocab: int, dmodel: int):
  n_total = n_batch * n_seq
  info = plsc.get_sparse_core_info()
  nw = info.num_cores * info.num_subcores
  assert n_total % nw == 0
  per_w = n_total // nw
  assert per_w % n_seq == 0
  rows_w = per_w // n_seq
  rows_c = 1
  chunk = rows_c * n_seq
  assert rows_w % rows_c == 0 and chunk % 8 == 0
  n_chunks = rows_w // rows_c

  mesh = plsc.VectorSubcoreMesh(core_axis_name="c", subcore_axis_name="s")

  @functools.partial(
      pl.kernel,
      mesh=mesh,
      out_type=jax.ShapeDtypeStruct((n_batch, n_seq, dmodel), jnp.float32),
      scratch_types=[
          pltpu.VMEM((per_w,), jnp.int32),
          pltpu.VMEM((2, rows_c, n_seq, _PAD), jnp.float32),
          pltpu.VMEM((2, rows_c, n_seq, dmodel), jnp.float32),
          pltpu.SemaphoreType.DMA,
          pltpu.SemaphoreType.DMA,
          pltpu.SemaphoreType.DMA,
          pltpu.SemaphoreType.DMA,
      ],
  )
  def k(idx_hbm, table_hbm, out3_hbm, idx_v, rows_v, wb_v, sg0, sg1, sw0, sw1):
    wid = lax.axis_index("s") * info.num_cores + lax.axis_index("c")
    base = wid * per_w
    pltpu.sync_copy(idx_hbm.at[pl.ds(base, per_w)], idx_v)

    sg = (sg0, sg1)
    sw = (sw0, sw1)

    def start_gather(g):
      return [
          pltpu.async_copy(
              table_hbm.at[idx_v.at[pl.ds(g * chunk + r * n_seq, n_seq)]],
              rows_v.at[g % 2, r],
              sg[g % 2],
          )
          for r in range(rows_c)
      ]

    def start_wb(g):
      b = g % 2
      def body(i, carry):
        for dj in range(2):
          s0 = i * 2 + dj
          for j in range(dmodel // 16):
            sl = pl.ds(j * 16, 16)
            wb_v[b, 0, s0, sl] = rows_v[b, 0, s0, sl]
        return carry

      lax.fori_loop(0, n_seq // 2, body, 0)
      return pltpu.async_copy(
          wb_v.at[b],
          out3_hbm.at[pl.ds(wid * rows_w + g * rows_c, rows_c)],
          sw[b],
      )

    def wait_gather(hs):
      for h in hs:
        h.wait()

    gh = [None] * n_chunks
    wh = [None] * n_chunks
    gh[0] = start_gather(0)
    for g in range(n_chunks):
      if g >= 2:
        wh[g - 2].wait()
      if g >= 1:
        gh[g] = start_gather(g)
        wait_gather(gh[g - 1])
        wh[g - 1] = start_wb(g - 1)
    wait_gather(gh[n_chunks - 1])
    wh[n_chunks - 1] = start_wb(n_chunks - 1)
    wh[n_chunks - 2].wait()
    wh[n_chunks - 1].wait()

  return k


def kernel(x, weight):
  b, s = x.shape
  vocab, dmodel = weight.shape
  flat_idx = x.reshape(-1).astype(jnp.int32)
  wp = jnp.pad(weight, ((0, 0), (0, _PAD - dmodel)))
  return _make_gather(b, s, vocab, dmodel)(flat_idx, wp)
